# baseline (speedup 1.0000x reference)
import functools

import jax
import jax.numpy as jnp
from jax import lax
from jax.experimental import pallas as pl
from jax.experimental.pallas import tpu as pltpu
from jax.experimental.pallas import tpu_sc as plsc

NC = 2
NS = 16
NW = NC * NS
CHUNK = 64
SB = 16
ROW_BLK = 1024


def _mesh():
  return plsc.VectorSubcoreMesh(core_axis_name="c", subcore_axis_name="s")


def _make_degree_kernel(npad, epad):
  epw = epad // NW
  chunks = epw // CHUNK
  rps = npad // NS

  @functools.partial(
      pl.kernel,
      out_type=(
          jax.ShapeDtypeStruct((NC, npad), jnp.float32),
          jax.ShapeDtypeStruct((NC, npad), jnp.float32),
      ),
      mesh=_mesh(),
      scratch_types=[
          pltpu.VMEM((chunks, CHUNK), jnp.int32),
          pltpu.VMEM((chunks, CHUNK), jnp.int32),
          pltpu.VMEM((CHUNK,), jnp.float32),
          pltpu.VMEM((rps,), jnp.float32),
          pltpu.VMEM_SHARED((npad,), jnp.float32),
          pltpu.VMEM_SHARED((npad,), jnp.float32),
          pltpu.SemaphoreType.DMA,
          pltpu.SemaphoreType.DMA,
      ],
  )
  def deg_kernel(src2_hbm, dst2_hbm, outd_hbm, ind_hbm, sidx, didx, ones,
                 zbuf, acc_o, acc_i, sem0, sem1):
    cid = lax.axis_index("c")
    sid = lax.axis_index("s")
    one = jnp.full((16,), 1.0, jnp.float32)
    zero = jnp.zeros((16,), jnp.float32)
    for k in range(CHUNK // 16):
      ones[pl.ds(k * 16, 16)] = one

    def zfill(i, _):
      zbuf[pl.ds(i * 16, 16)] = zero
      return 0

    lax.fori_loop(0, rps // 16, zfill, 0, unroll=False)
    pltpu.sync_copy(zbuf, acc_o.at[pl.ds(sid * rps, rps)])
    pltpu.sync_copy(zbuf, acc_i.at[pl.ds(sid * rps, rps)])

    rowbase = (cid * NS + sid) * chunks
    pltpu.sync_copy(src2_hbm.at[pl.ds(rowbase, chunks)], sidx)
    pltpu.sync_copy(dst2_hbm.at[pl.ds(rowbase, chunks)], didx)
    plsc.subcore_barrier()

    def body(g, _):
      pltpu.async_copy(ones, acc_o.at[sidx.at[g]], sem0, add=True)
      pltpu.async_copy(ones, acc_i.at[didx.at[g]], sem1, add=True)
      return 0

    lax.fori_loop(0, chunks, body, 0, unroll=False)

    def drain(g, _):
      pltpu.make_async_copy(ones, acc_o.at[sidx.at[0]], sem0).wait()
      pltpu.make_async_copy(ones, acc_i.at[didx.at[0]], sem1).wait()
      return 0

    lax.fori_loop(0, chunks, drain, 0, unroll=False)
    plsc.subcore_barrier()
    pltpu.sync_copy(acc_o.at[pl.ds(sid * rps, rps)],
                    outd_hbm.at[cid, pl.ds(sid * rps, rps)])
    pltpu.sync_copy(acc_i.at[pl.ds(sid * rps, rps)],
                    ind_hbm.at[cid, pl.ds(sid * rps, rps)])

  return deg_kernel


def _make_agg_kernel(npad, epad, d):
  epw = epad // NW
  chunks = epw // CHUNK
  rps = npad // NS

  @functools.partial(
      pl.kernel,
      out_type=jax.ShapeDtypeStruct((NC, npad, d), jnp.float32),
      mesh=_mesh(),
      scratch_types=[
          pltpu.VMEM((SB, CHUNK), jnp.int32),
          pltpu.VMEM((SB, CHUNK), jnp.int32),
          pltpu.VMEM((4, CHUNK, d), jnp.float32),
          pltpu.VMEM((16, d), jnp.float32),
          pltpu.VMEM_SHARED((npad, d), jnp.float32),
          [pltpu.SemaphoreType.DMA] * 4,
          [pltpu.SemaphoreType.DMA] * 4,
      ],
  )
  def agg_kernel(t_hbm, src2_hbm, dst2_hbm, out_hbm, sidx, didx, rows, zbuf,
                 acc, gsems, ssems):
    cid = lax.axis_index("c")
    sid = lax.axis_index("s")
    zero = jnp.zeros((16,), jnp.float32)
    for r in range(16):
      for j in range(d // 16):
        zbuf[r, pl.ds(j * 16, 16)] = zero

    def zero_body(i, _):
      pltpu.sync_copy(zbuf, acc.at[pl.ds(sid * rps + i * 16, 16)])
      return 0

    lax.fori_loop(0, rps // 16, zero_body, 0, unroll=False)
    plsc.subcore_barrier()

    rowbase = (cid * NS + sid) * chunks

    def fire_g(c, b):
      pltpu.async_copy(t_hbm.at[sidx.at[c]], rows.at[b], gsems[b])

    def wait_g(b):
      pltpu.make_async_copy(t_hbm.at[sidx.at[0]], rows.at[b],
                            gsems[b]).wait()

    def fire_s(c, b):
      pltpu.async_copy(rows.at[b], acc.at[didx.at[c]], ssems[b], add=True)

    def wait_s(b):
      pltpu.make_async_copy(rows.at[b], acc.at[didx.at[0]], ssems[b]).wait()

    def sb_body(sb, _):
      rowb = rowbase + sb * SB
      pltpu.sync_copy(src2_hbm.at[pl.ds(rowb, SB)], sidx)
      pltpu.sync_copy(dst2_hbm.at[pl.ds(rowb, SB)], didx)
      fire_g(0, 0)
      fire_g(1, 1)
      wait_g(0)
      fire_s(0, 0)
      fire_g(2, 2)
      wait_g(1)
      fire_s(1, 1)
      fire_g(3, 3)

      def body(j, _):
        c0 = 4 * j + 2
        wait_g(2)
        fire_s(c0, 2)
        wait_s(0)
        fire_g(c0 + 2, 0)
        wait_g(3)
        fire_s(c0 + 1, 3)
        wait_s(1)
        fire_g(c0 + 3, 1)
        wait_g(0)
        fire_s(c0 + 2, 0)
        wait_s(2)
        fire_g(c0 + 4, 2)
        wait_g(1)
        fire_s(c0 + 3, 1)
        wait_s(3)
        fire_g(c0 + 5, 3)
        return 0

      lax.fori_loop(0, (SB - 4) // 4, body, 0, unroll=False)
      wait_g(2)
      fire_s(SB - 2, 2)
      wait_s(0)
      wait_g(3)
      fire_s(SB - 1, 3)
      wait_s(1)
      wait_s(2)
      wait_s(3)
      return 0

    lax.fori_loop(0, chunks // SB, sb_body, 0, unroll=False)
    plsc.subcore_barrier()
    pltpu.sync_copy(acc.at[pl.ds(sid * rps, rps)],
                    out_hbm.at[cid, pl.ds(sid * rps, rps)])

  return agg_kernel


def _proj_body(xr_ref, wp_ref, bp_ref, o_ref):
  x = jnp.dot(xr_ref[...], wp_ref[...], preferred_element_type=jnp.float32)
  o_ref[...] = jnp.maximum(x + bp_ref[...], 0.0)


def _norm(d_ref):
  deg = jnp.sum(d_ref[...], axis=0)
  return lax.rsqrt(jnp.maximum(deg, 1.0))[:, None]


def _scale_mm_body(x_ref, do_ref, w_ref, o_ref):
  o_ref[...] = jnp.dot(x_ref[...] * _norm(do_ref), w_ref[...],
                       preferred_element_type=jnp.float32)


def _mid_body(a0_ref, a1_ref, di_ref, do_ref, b_ref, w_ref, o_ref):
  h = jnp.maximum((a0_ref[...] + a1_ref[...]) * _norm(di_ref) + b_ref[...],
                  0.0)
  o_ref[...] = jnp.dot(h * _norm(do_ref), w_ref[...],
                       preferred_element_type=jnp.float32)


def _final_body(a0_ref, a1_ref, di_ref, b_ref, o_ref):
  o_ref[...] = jnp.maximum(
      (a0_ref[...] + a1_ref[...]) * _norm(di_ref) + b_ref[...], 0.0)


def _row_spec(d):
  return pl.BlockSpec((ROW_BLK, d), lambda i: (i, 0))


def _full_spec(shape):
  return pl.BlockSpec(shape, lambda i: tuple(0 for _ in shape))


def kernel(x_raw, edge_index, Wp, bp, W1, b1, W2, b2):
  n, in_dim = x_raw.shape
  hid = W1.shape[1]
  e = edge_index.shape[1]

  npad = ((n + ROW_BLK) // ROW_BLK) * ROW_BLK
  egrain = NW * CHUNK * SB
  epad = ((e + egrain - 1) // egrain) * egrain
  n_dummy = npad - n

  src = edge_index[0].astype(jnp.int32)
  dst = edge_index[1].astype(jnp.int32)
  pad_idx = (jnp.arange(epad - e, dtype=jnp.int32) % n_dummy) + n
  src = jnp.concatenate([src, pad_idx]).reshape(epad // CHUNK, CHUNK)
  dst = jnp.concatenate([dst, pad_idx]).reshape(epad // CHUNK, CHUNK)
  xp = jnp.pad(x_raw, ((0, npad - n), (0, 0)))

  grid = (npad // ROW_BLK,)

  deg_kernel = _make_degree_kernel(npad, epad)
  agg_kernel = _make_agg_kernel(npad, epad, hid)

  outd_p, ind_p = deg_kernel(src, dst)

  bp2 = bp.reshape(1, hid)
  b12 = b1.reshape(1, hid)
  b22 = b2.reshape(1, hid)
  dspec = pl.BlockSpec((NC, ROW_BLK), lambda i: (0, i))
  row = _row_spec(hid)
  wspec = _full_spec((in_dim, hid))
  bspec = _full_spec((1, hid))
  oshape = jax.ShapeDtypeStruct((npad, hid), jnp.float32)

  x = pl.pallas_call(
      _proj_body, grid=grid,
      in_specs=[pl.BlockSpec((ROW_BLK, in_dim), lambda i: (i, 0)), wspec,
                bspec],
      out_specs=row, out_shape=oshape,
  )(xp, Wp, bp2)

  t1 = pl.pallas_call(
      _scale_mm_body, grid=grid,
      in_specs=[row, dspec, wspec],
      out_specs=row, out_shape=oshape,
  )(x, outd_p, W1)

  agg1 = agg_kernel(t1, src, dst)

  t2 = pl.pallas_call(
      _mid_body, grid=grid,
      in_specs=[row, row, dspec, dspec, bspec, wspec],
      out_specs=row, out_shape=oshape,
  )(agg1[0], agg1[1], ind_p, outd_p, b12, W2)

  agg2 = agg_kernel(t2, src, dst)

  out = pl.pallas_call(
      _final_body, grid=grid,
      in_specs=[row, row, dspec, bspec],
      out_specs=row, out_shape=oshape,
  )(agg2[0], agg2[1], ind_p, b22)

  return out[:n]


# speedup vs baseline: 1.1280x; 1.1280x over previous
import functools

import jax
import jax.numpy as jnp
from jax import lax
from jax.experimental import pallas as pl
from jax.experimental.pallas import tpu as pltpu
from jax.experimental.pallas import tpu_sc as plsc

NC = 2
NS = 16
NW = NC * NS
CHUNK = 128
SB = 8
ROW_BLK = 1024


def _mesh():
  return plsc.VectorSubcoreMesh(core_axis_name="c", subcore_axis_name="s")


def _make_degree_kernel(npad, epad):
  epw = epad // NW
  chunks = epw // CHUNK
  rps = npad // NS

  @functools.partial(
      pl.kernel,
      out_type=(
          jax.ShapeDtypeStruct((NC, npad), jnp.float32),
          jax.ShapeDtypeStruct((NC, npad), jnp.float32),
      ),
      mesh=_mesh(),
      scratch_types=[
          pltpu.VMEM((chunks, CHUNK), jnp.int32),
          pltpu.VMEM((chunks, CHUNK), jnp.int32),
          pltpu.VMEM((CHUNK,), jnp.float32),
          pltpu.VMEM((rps,), jnp.float32),
          pltpu.VMEM_SHARED((npad,), jnp.float32),
          pltpu.VMEM_SHARED((npad,), jnp.float32),
          pltpu.SemaphoreType.DMA,
          pltpu.SemaphoreType.DMA,
      ],
  )
  def deg_kernel(src2_hbm, dst2_hbm, outd_hbm, ind_hbm, sidx, didx, ones,
                 zbuf, acc_o, acc_i, sem0, sem1):
    cid = lax.axis_index("c")
    sid = lax.axis_index("s")
    one = jnp.full((16,), 1.0, jnp.float32)
    zero = jnp.zeros((16,), jnp.float32)
    for k in range(CHUNK // 16):
      ones[pl.ds(k * 16, 16)] = one

    def zfill(i, _):
      zbuf[pl.ds(i * 16, 16)] = zero
      return 0

    lax.fori_loop(0, rps // 16, zfill, 0, unroll=False)
    pltpu.sync_copy(zbuf, acc_o.at[pl.ds(sid * rps, rps)])
    pltpu.sync_copy(zbuf, acc_i.at[pl.ds(sid * rps, rps)])

    rowbase = (cid * NS + sid) * chunks
    pltpu.sync_copy(src2_hbm.at[pl.ds(rowbase, chunks)], sidx)
    pltpu.sync_copy(dst2_hbm.at[pl.ds(rowbase, chunks)], didx)
    plsc.subcore_barrier()

    def body(g, _):
      pltpu.async_copy(ones, acc_o.at[sidx.at[g]], sem0, add=True)
      pltpu.async_copy(ones, acc_i.at[didx.at[g]], sem1, add=True)
      return 0

    lax.fori_loop(0, chunks, body, 0, unroll=False)

    def drain(g, _):
      pltpu.make_async_copy(ones, acc_o.at[sidx.at[0]], sem0).wait()
      pltpu.make_async_copy(ones, acc_i.at[didx.at[0]], sem1).wait()
      return 0

    lax.fori_loop(0, chunks, drain, 0, unroll=False)
    plsc.subcore_barrier()
    pltpu.sync_copy(acc_o.at[pl.ds(sid * rps, rps)],
                    outd_hbm.at[cid, pl.ds(sid * rps, rps)])
    pltpu.sync_copy(acc_i.at[pl.ds(sid * rps, rps)],
                    ind_hbm.at[cid, pl.ds(sid * rps, rps)])

  return deg_kernel


def _make_agg_kernel(npad, epad, d):
  epw = epad // NW
  chunks = epw // CHUNK
  rps = npad // NS

  @functools.partial(
      pl.kernel,
      out_type=jax.ShapeDtypeStruct((NC, npad, d), jnp.float32),
      mesh=_mesh(),
      scratch_types=[
          pltpu.VMEM((2, SB, CHUNK), jnp.int32),
          pltpu.VMEM((2, SB, CHUNK), jnp.int32),
          pltpu.VMEM((2, CHUNK, d), jnp.float32),
          pltpu.VMEM((16, d), jnp.float32),
          pltpu.VMEM_SHARED((npad, d), jnp.float32),
          pltpu.SemaphoreType.DMA,
          pltpu.SemaphoreType.DMA,
          pltpu.SemaphoreType.DMA,
      ],
  )
  def agg_kernel(t_hbm, src2_hbm, dst2_hbm, out_hbm, sidx, didx, rows, zbuf,
                 acc, sem0, sem1, isem):
    cid = lax.axis_index("c")
    sid = lax.axis_index("s")
    zero = jnp.zeros((16,), jnp.float32)
    for r in range(16):
      for j in range(d // 16):
        zbuf[r, pl.ds(j * 16, 16)] = zero

    def zero_body(i, _):
      pltpu.sync_copy(zbuf, acc.at[pl.ds(sid * rps + i * 16, 16)])
      return 0

    lax.fori_loop(0, rps // 16, zero_body, 0, unroll=False)
    plsc.subcore_barrier()

    rowbase = (cid * NS + sid) * chunks
    sems = (sem0, sem1)

    def fire_idx(sb, p):
      rowb = rowbase + sb * SB
      pltpu.async_copy(src2_hbm.at[pl.ds(rowb, SB)], sidx.at[p], isem)
      pltpu.async_copy(dst2_hbm.at[pl.ds(rowb, SB)], didx.at[p], isem)

    def wait_idx(p):
      pltpu.make_async_copy(src2_hbm.at[pl.ds(0, SB)], sidx.at[p],
                            isem).wait()
      pltpu.make_async_copy(dst2_hbm.at[pl.ds(0, SB)], didx.at[p],
                            isem).wait()

    def fire(p, c, b):
      pltpu.async_copy(t_hbm.at[sidx.at[p, c]], rows.at[b], sems[b])

    def wait(b):
      pltpu.make_async_copy(t_hbm.at[sidx.at[0, 0]], rows.at[b],
                            sems[b]).wait()

    def scat(p, c, b):
      pltpu.sync_copy(rows.at[b], acc.at[didx.at[p, c]], add=True)

    def run_sb(sb, p):
      wait_idx(p)
      fire_idx(sb + 1, 1 - p)
      fire(p, 0, 0)
      fire(p, 1, 1)

      def body(j, _):
        c0 = 2 * j
        wait(0)
        scat(p, c0, 0)
        fire(p, c0 + 2, 0)
        wait(1)
        scat(p, c0 + 1, 1)
        fire(p, c0 + 3, 1)
        return 0

      lax.fori_loop(0, (SB - 2) // 2, body, 0, unroll=False)
      wait(0)
      scat(p, SB - 2, 0)
      wait(1)
      scat(p, SB - 1, 1)

    fire_idx(0, 0)

    def sb_pair(i, _):
      run_sb(2 * i, 0)
      run_sb(2 * i + 1, 1)
      return 0

    lax.fori_loop(0, chunks // SB // 2, sb_pair, 0, unroll=False)
    wait_idx(0)
    plsc.subcore_barrier()
    pltpu.sync_copy(acc.at[pl.ds(sid * rps, rps)],
                    out_hbm.at[cid, pl.ds(sid * rps, rps)])

  return agg_kernel


def _proj_body(xr_ref, wp_ref, bp_ref, o_ref):
  x = jnp.dot(xr_ref[...], wp_ref[...], preferred_element_type=jnp.float32)
  o_ref[...] = jnp.maximum(x + bp_ref[...], 0.0)


def _norm(d_ref):
  deg = jnp.sum(d_ref[...], axis=0)
  return lax.rsqrt(jnp.maximum(deg, 1.0))[:, None]


def _scale_mm_body(x_ref, do_ref, w_ref, o_ref):
  o_ref[...] = jnp.dot(x_ref[...] * _norm(do_ref), w_ref[...],
                       preferred_element_type=jnp.float32)


def _mid_body(a0_ref, a1_ref, di_ref, do_ref, b_ref, w_ref, o_ref):
  h = jnp.maximum((a0_ref[...] + a1_ref[...]) * _norm(di_ref) + b_ref[...],
                  0.0)
  o_ref[...] = jnp.dot(h * _norm(do_ref), w_ref[...],
                       preferred_element_type=jnp.float32)


def _final_body(a0_ref, a1_ref, di_ref, b_ref, o_ref):
  o_ref[...] = jnp.maximum(
      (a0_ref[...] + a1_ref[...]) * _norm(di_ref) + b_ref[...], 0.0)


def _row_spec(d):
  return pl.BlockSpec((ROW_BLK, d), lambda i: (i, 0))


def _full_spec(shape):
  return pl.BlockSpec(shape, lambda i: tuple(0 for _ in shape))


def kernel(x_raw, edge_index, Wp, bp, W1, b1, W2, b2):
  n, in_dim = x_raw.shape
  hid = W1.shape[1]
  e = edge_index.shape[1]

  npad = ((n + ROW_BLK) // ROW_BLK) * ROW_BLK
  egrain = NW * CHUNK * SB * 2
  epad = ((e + egrain - 1) // egrain) * egrain
  n_dummy = npad - n

  src = edge_index[0].astype(jnp.int32)
  dst = edge_index[1].astype(jnp.int32)
  pad_idx = (jnp.arange(epad - e + SB * CHUNK, dtype=jnp.int32) % n_dummy) + n
  src = jnp.concatenate([src, pad_idx]).reshape(-1, CHUNK)
  dst = jnp.concatenate([dst, pad_idx]).reshape(-1, CHUNK)
  xp = jnp.pad(x_raw, ((0, npad - n), (0, 0)))

  grid = (npad // ROW_BLK,)

  deg_kernel = _make_degree_kernel(npad, epad)
  agg_kernel = _make_agg_kernel(npad, epad, hid)

  outd_p, ind_p = deg_kernel(src, dst)

  bp2 = bp.reshape(1, hid)
  b12 = b1.reshape(1, hid)
  b22 = b2.reshape(1, hid)
  dspec = pl.BlockSpec((NC, ROW_BLK), lambda i: (0, i))
  row = _row_spec(hid)
  wspec = _full_spec((in_dim, hid))
  bspec = _full_spec((1, hid))
  oshape = jax.ShapeDtypeStruct((npad, hid), jnp.float32)

  x = pl.pallas_call(
      _proj_body, grid=grid,
      in_specs=[pl.BlockSpec((ROW_BLK, in_dim), lambda i: (i, 0)), wspec,
                bspec],
      out_specs=row, out_shape=oshape,
  )(xp, Wp, bp2)

  t1 = pl.pallas_call(
      _scale_mm_body, grid=grid,
      in_specs=[row, dspec, wspec],
      out_specs=row, out_shape=oshape,
  )(x, outd_p, W1)

  agg1 = agg_kernel(t1, src, dst)

  t2 = pl.pallas_call(
      _mid_body, grid=grid,
      in_specs=[row, row, dspec, dspec, bspec, wspec],
      out_specs=row, out_shape=oshape,
  )(agg1[0], agg1[1], ind_p, outd_p, b12, W2)

  agg2 = agg_kernel(t2, src, dst)

  out = pl.pallas_call(
      _final_body, grid=grid,
      in_specs=[row, row, dspec, bspec],
      out_specs=row, out_shape=oshape,
  )(agg2[0], agg2[1], ind_p, b22)

  return out[:n]


# speedup vs baseline: 1.1303x; 1.0020x over previous
import functools

import jax
import jax.numpy as jnp
from jax import lax
from jax.experimental import pallas as pl
from jax.experimental.pallas import tpu as pltpu
from jax.experimental.pallas import tpu_sc as plsc

NC = 2
NS = 16
NW = NC * NS
CHUNK = 128
SB = 8
ROW_BLK = 1024


def _mesh():
  return plsc.VectorSubcoreMesh(core_axis_name="c", subcore_axis_name="s")


def _make_degree_kernel(npad, epad):
  epw = epad // NW
  chunks = epw // CHUNK
  rps = npad // NS

  @functools.partial(
      pl.kernel,
      out_type=(
          jax.ShapeDtypeStruct((NC, npad), jnp.float32),
          jax.ShapeDtypeStruct((NC, npad), jnp.float32),
      ),
      mesh=_mesh(),
      scratch_types=[
          pltpu.VMEM((chunks, CHUNK), jnp.int32),
          pltpu.VMEM((chunks, CHUNK), jnp.int32),
          pltpu.VMEM((CHUNK,), jnp.float32),
          pltpu.VMEM((rps,), jnp.float32),
          pltpu.VMEM_SHARED((npad,), jnp.float32),
          pltpu.VMEM_SHARED((npad,), jnp.float32),
          pltpu.SemaphoreType.DMA,
          pltpu.SemaphoreType.DMA,
      ],
  )
  def deg_kernel(src2_hbm, dst2_hbm, outd_hbm, ind_hbm, sidx, didx, ones,
                 zbuf, acc_o, acc_i, sem0, sem1):
    cid = lax.axis_index("c")
    sid = lax.axis_index("s")
    one = jnp.full((16,), 1.0, jnp.float32)
    zero = jnp.zeros((16,), jnp.float32)
    for k in range(CHUNK // 16):
      ones[pl.ds(k * 16, 16)] = one

    def zfill(i, _):
      zbuf[pl.ds(i * 16, 16)] = zero
      return 0

    lax.fori_loop(0, rps // 16, zfill, 0, unroll=False)
    pltpu.sync_copy(zbuf, acc_o.at[pl.ds(sid * rps, rps)])
    pltpu.sync_copy(zbuf, acc_i.at[pl.ds(sid * rps, rps)])

    rowbase = (cid * NS + sid) * chunks
    pltpu.sync_copy(src2_hbm.at[pl.ds(rowbase, chunks)], sidx)
    pltpu.sync_copy(dst2_hbm.at[pl.ds(rowbase, chunks)], didx)
    plsc.subcore_barrier()

    def body(g, _):
      pltpu.async_copy(ones, acc_o.at[sidx.at[g]], sem0, add=True)
      pltpu.async_copy(ones, acc_i.at[didx.at[g]], sem1, add=True)
      return 0

    lax.fori_loop(0, chunks, body, 0, unroll=False)

    def drain(g, _):
      pltpu.make_async_copy(ones, acc_o.at[sidx.at[0]], sem0).wait()
      pltpu.make_async_copy(ones, acc_i.at[didx.at[0]], sem1).wait()
      return 0

    lax.fori_loop(0, chunks, drain, 0, unroll=False)
    plsc.subcore_barrier()
    pltpu.sync_copy(acc_o.at[pl.ds(sid * rps, rps)],
                    outd_hbm.at[cid, pl.ds(sid * rps, rps)])
    pltpu.sync_copy(acc_i.at[pl.ds(sid * rps, rps)],
                    ind_hbm.at[cid, pl.ds(sid * rps, rps)])

  return deg_kernel


def _make_agg_kernel(npad, epad, d):
  epw = epad // NW
  chunks = epw // CHUNK
  rps = npad // NS

  @functools.partial(
      pl.kernel,
      out_type=jax.ShapeDtypeStruct((NC, npad, d), jnp.float32),
      mesh=_mesh(),
      scratch_types=[
          pltpu.VMEM((2, SB, CHUNK), jnp.int32),
          pltpu.VMEM((2, SB, CHUNK), jnp.int32),
          pltpu.VMEM((2, CHUNK, d), jnp.float32),
          pltpu.VMEM((16, d), jnp.float32),
          pltpu.VMEM_SHARED((npad, d), jnp.float32),
          pltpu.SemaphoreType.DMA,
          pltpu.SemaphoreType.DMA,
          pltpu.SemaphoreType.DMA,
      ],
  )
  def agg_kernel(t_hbm, src2_hbm, dst2_hbm, out_hbm, sidx, didx, rows, zbuf,
                 acc, sem0, sem1, isem):
    cid = lax.axis_index("c")
    sid = lax.axis_index("s")
    zero = jnp.zeros((16,), jnp.float32)
    for r in range(16):
      for j in range(d // 16):
        zbuf[r, pl.ds(j * 16, 16)] = zero

    def zero_body(i, _):
      pltpu.sync_copy(zbuf, acc.at[pl.ds(sid * rps + i * 16, 16)])
      return 0

    lax.fori_loop(0, rps // 16, zero_body, 0, unroll=False)
    plsc.subcore_barrier()

    rowbase = (cid * NS + sid) * chunks
    sems = (sem0, sem1)

    def fire_idx(sb, p):
      rowb = rowbase + sb * SB
      pltpu.async_copy(src2_hbm.at[pl.ds(rowb, SB)], sidx.at[p], isem)
      pltpu.async_copy(dst2_hbm.at[pl.ds(rowb, SB)], didx.at[p], isem)

    def wait_idx(p):
      pltpu.make_async_copy(src2_hbm.at[pl.ds(0, SB)], sidx.at[p],
                            isem).wait()
      pltpu.make_async_copy(dst2_hbm.at[pl.ds(0, SB)], didx.at[p],
                            isem).wait()

    def fire(p, c, b):
      pltpu.async_copy(t_hbm.at[sidx.at[p, c]], rows.at[b], sems[b])

    def wait(b):
      pltpu.make_async_copy(t_hbm.at[sidx.at[0, 0]], rows.at[b],
                            sems[b]).wait()

    def scat(p, c, b):
      pltpu.sync_copy(rows.at[b], acc.at[didx.at[p, c]], add=True)

    def run_sb(sb, p):
      wait_idx(p)
      fire_idx(sb + 1, 1 - p)
      fire(p, 0, 0)
      fire(p, 1, 1)

      def body(j, _):
        c0 = 2 * j
        wait(0)
        scat(p, c0, 0)
        fire(p, c0 + 2, 0)
        wait(1)
        scat(p, c0 + 1, 1)
        fire(p, c0 + 3, 1)
        return 0

      lax.fori_loop(0, (SB - 2) // 2, body, 0, unroll=False)
      wait(0)
      scat(p, SB - 2, 0)
      wait(1)
      scat(p, SB - 1, 1)

    fire_idx(0, 0)

    def sb_pair(i, _):
      run_sb(2 * i, 0)
      run_sb(2 * i + 1, 1)
      return 0

    lax.fori_loop(0, chunks // SB // 2, sb_pair, 0, unroll=False)
    wait_idx(0)
    plsc.subcore_barrier()
    pltpu.sync_copy(acc.at[pl.ds(sid * rps, rps)],
                    out_hbm.at[cid, pl.ds(sid * rps, rps)])

  return agg_kernel


def _proj_scale_mm_body(xr_ref, wp_ref, bp_ref, do_ref, w1_ref, o_ref):
  x = jnp.dot(xr_ref[...], wp_ref[...], preferred_element_type=jnp.float32)
  x = jnp.maximum(x + bp_ref[...], 0.0)
  o_ref[...] = jnp.dot(x * _norm(do_ref), w1_ref[...],
                       preferred_element_type=jnp.float32)


def _norm(d_ref):
  deg = jnp.sum(d_ref[...], axis=0)
  return lax.rsqrt(jnp.maximum(deg, 1.0))[:, None]


def _mid_body(a0_ref, a1_ref, di_ref, do_ref, b_ref, w_ref, o_ref):
  h = jnp.maximum((a0_ref[...] + a1_ref[...]) * _norm(di_ref) + b_ref[...],
                  0.0)
  o_ref[...] = jnp.dot(h * _norm(do_ref), w_ref[...],
                       preferred_element_type=jnp.float32)


def _final_body(a0_ref, a1_ref, di_ref, b_ref, o_ref):
  o_ref[...] = jnp.maximum(
      (a0_ref[...] + a1_ref[...]) * _norm(di_ref) + b_ref[...], 0.0)


def _row_spec(d):
  return pl.BlockSpec((ROW_BLK, d), lambda i: (i, 0))


def _full_spec(shape):
  return pl.BlockSpec(shape, lambda i: tuple(0 for _ in shape))


def kernel(x_raw, edge_index, Wp, bp, W1, b1, W2, b2):
  n, in_dim = x_raw.shape
  hid = W1.shape[1]
  e = edge_index.shape[1]

  npad = ((n + ROW_BLK) // ROW_BLK) * ROW_BLK
  egrain = NW * CHUNK * SB * 2
  epad = ((e + egrain - 1) // egrain) * egrain
  n_dummy = npad - n

  src = edge_index[0].astype(jnp.int32)
  dst = edge_index[1].astype(jnp.int32)
  pad_idx = (jnp.arange(epad - e + SB * CHUNK, dtype=jnp.int32) % n_dummy) + n
  src = jnp.concatenate([src, pad_idx]).reshape(-1, CHUNK)
  dst = jnp.concatenate([dst, pad_idx]).reshape(-1, CHUNK)
  xp = jnp.pad(x_raw, ((0, npad - n), (0, 0)))

  grid = (npad // ROW_BLK,)

  deg_kernel = _make_degree_kernel(npad, epad)
  agg_kernel = _make_agg_kernel(npad, epad, hid)

  outd_p, ind_p = deg_kernel(src, dst)

  bp2 = bp.reshape(1, hid)
  b12 = b1.reshape(1, hid)
  b22 = b2.reshape(1, hid)
  dspec = pl.BlockSpec((NC, ROW_BLK), lambda i: (0, i))
  row = _row_spec(hid)
  wspec = _full_spec((in_dim, hid))
  bspec = _full_spec((1, hid))
  oshape = jax.ShapeDtypeStruct((npad, hid), jnp.float32)

  t1 = pl.pallas_call(
      _proj_scale_mm_body, grid=grid,
      in_specs=[pl.BlockSpec((ROW_BLK, in_dim), lambda i: (i, 0)), wspec,
                bspec, dspec, wspec],
      out_specs=row, out_shape=oshape,
  )(xp, Wp, bp2, outd_p, W1)

  agg1 = agg_kernel(t1, src, dst)

  t2 = pl.pallas_call(
      _mid_body, grid=grid,
      in_specs=[row, row, dspec, dspec, bspec, wspec],
      out_specs=row, out_shape=oshape,
  )(agg1[0], agg1[1], ind_p, outd_p, b12, W2)

  agg2 = agg_kernel(t2, src, dst)

  out = pl.pallas_call(
      _final_body, grid=grid,
      in_specs=[row, row, dspec, bspec],
      out_specs=row, out_shape=oshape,
  )(agg2[0], agg2[1], ind_p, b22)

  return out[:n]


# speedup vs baseline: 1.1430x; 1.0112x over previous
import functools

import jax
import jax.numpy as jnp
from jax import lax
from jax.experimental import pallas as pl
from jax.experimental.pallas import tpu as pltpu
from jax.experimental.pallas import tpu_sc as plsc

NC = 2
NS = 16
NW = NC * NS
CHUNK = 128
SB = 8
ROW_BLK = 1024


def _mesh():
  return plsc.VectorSubcoreMesh(core_axis_name="c", subcore_axis_name="s")


def _make_degree_kernel(npad, epad):
  epw = epad // NW
  chunks = epw // CHUNK
  rps = npad // NS

  @functools.partial(
      pl.kernel,
      out_type=(
          jax.ShapeDtypeStruct((NC, npad), jnp.float32),
          jax.ShapeDtypeStruct((NC, npad), jnp.float32),
      ),
      mesh=_mesh(),
      scratch_types=[
          pltpu.VMEM((chunks, CHUNK), jnp.int32),
          pltpu.VMEM((chunks, CHUNK), jnp.int32),
          pltpu.VMEM((CHUNK,), jnp.float32),
          pltpu.VMEM((rps,), jnp.float32),
          pltpu.VMEM_SHARED((npad,), jnp.float32),
          pltpu.VMEM_SHARED((npad,), jnp.float32),
          pltpu.SemaphoreType.DMA,
          pltpu.SemaphoreType.DMA,
      ],
  )
  def deg_kernel(src2_hbm, dst2_hbm, outd_hbm, ind_hbm, sidx, didx, ones,
                 zbuf, acc_o, acc_i, sem0, sem1):
    cid = lax.axis_index("c")
    sid = lax.axis_index("s")
    one = jnp.full((16,), 1.0, jnp.float32)
    zero = jnp.zeros((16,), jnp.float32)
    for k in range(CHUNK // 16):
      ones[pl.ds(k * 16, 16)] = one

    def zfill(i, _):
      zbuf[pl.ds(i * 16, 16)] = zero
      return 0

    lax.fori_loop(0, rps // 16, zfill, 0, unroll=False)
    pltpu.sync_copy(zbuf, acc_o.at[pl.ds(sid * rps, rps)])
    pltpu.sync_copy(zbuf, acc_i.at[pl.ds(sid * rps, rps)])

    rowbase = (cid * NS + sid) * chunks
    pltpu.sync_copy(src2_hbm.at[pl.ds(rowbase, chunks)], sidx)
    pltpu.sync_copy(dst2_hbm.at[pl.ds(rowbase, chunks)], didx)
    plsc.subcore_barrier()

    def body(g, _):
      pltpu.async_copy(ones, acc_o.at[sidx.at[g]], sem0, add=True)
      pltpu.async_copy(ones, acc_i.at[didx.at[g]], sem1, add=True)
      return 0

    lax.fori_loop(0, chunks, body, 0, unroll=False)

    def drain(g, _):
      pltpu.make_async_copy(ones, acc_o.at[sidx.at[0]], sem0).wait()
      pltpu.make_async_copy(ones, acc_i.at[didx.at[0]], sem1).wait()
      return 0

    lax.fori_loop(0, chunks, drain, 0, unroll=False)
    plsc.subcore_barrier()
    pltpu.sync_copy(acc_o.at[pl.ds(sid * rps, rps)],
                    outd_hbm.at[cid, pl.ds(sid * rps, rps)])
    pltpu.sync_copy(acc_i.at[pl.ds(sid * rps, rps)],
                    ind_hbm.at[cid, pl.ds(sid * rps, rps)])

  return deg_kernel


def _make_agg_kernel(npad, epad, d):
  epw = epad // NW
  chunks = epw // CHUNK
  rps = npad // NS

  @functools.partial(
      pl.kernel,
      out_type=jax.ShapeDtypeStruct((NC, npad, d), jnp.float32),
      mesh=_mesh(),
      scratch_types=[
          pltpu.VMEM((2, SB, CHUNK), jnp.int32),
          pltpu.VMEM((2, SB, CHUNK), jnp.int32),
          pltpu.VMEM((2, CHUNK, d), jnp.float32),
          pltpu.VMEM((64, d), jnp.float32),
          pltpu.VMEM_SHARED((npad, d), jnp.float32),
          pltpu.SemaphoreType.DMA,
          pltpu.SemaphoreType.DMA,
          pltpu.SemaphoreType.DMA,
      ],
  )
  def agg_kernel(t_hbm, src2_hbm, dst2_hbm, out_hbm, sidx, didx, rows, zbuf,
                 acc, sem0, sem1, isem):
    cid = lax.axis_index("c")
    sid = lax.axis_index("s")
    zero = jnp.zeros((16,), jnp.float32)

    def zfill(r, _):
      for j in range(d // 16):
        zbuf[r, pl.ds(j * 16, 16)] = zero
      return 0

    lax.fori_loop(0, 64, zfill, 0, unroll=False)

    def zero_fire(i, _):
      pltpu.async_copy(zbuf, acc.at[pl.ds(sid * rps + i * 64, 64)], isem)
      return 0

    def zero_drain(i, _):
      pltpu.make_async_copy(zbuf, acc.at[pl.ds(sid * rps, 64)], isem).wait()
      return 0

    lax.fori_loop(0, rps // 64, zero_fire, 0, unroll=False)
    lax.fori_loop(0, rps // 64, zero_drain, 0, unroll=False)
    plsc.subcore_barrier()

    rowbase = (cid * NS + sid) * chunks
    sems = (sem0, sem1)

    def fire_idx(sb, p):
      rowb = rowbase + sb * SB
      pltpu.async_copy(src2_hbm.at[pl.ds(rowb, SB)], sidx.at[p], isem)
      pltpu.async_copy(dst2_hbm.at[pl.ds(rowb, SB)], didx.at[p], isem)

    def wait_idx(p):
      pltpu.make_async_copy(src2_hbm.at[pl.ds(0, SB)], sidx.at[p],
                            isem).wait()
      pltpu.make_async_copy(dst2_hbm.at[pl.ds(0, SB)], didx.at[p],
                            isem).wait()

    def fire(p, c, b):
      pltpu.async_copy(t_hbm.at[sidx.at[p, c]], rows.at[b], sems[b])

    def wait(b):
      pltpu.make_async_copy(t_hbm.at[sidx.at[0, 0]], rows.at[b],
                            sems[b]).wait()

    def scat(p, c, b):
      pltpu.sync_copy(rows.at[b], acc.at[didx.at[p, c]], add=True)

    def run_sb(sb, p):
      wait_idx(p)
      fire_idx(sb + 1, 1 - p)
      fire(p, 0, 0)
      fire(p, 1, 1)

      def body(j, _):
        c0 = 2 * j
        wait(0)
        scat(p, c0, 0)
        fire(p, c0 + 2, 0)
        wait(1)
        scat(p, c0 + 1, 1)
        fire(p, c0 + 3, 1)
        return 0

      lax.fori_loop(0, (SB - 2) // 2, body, 0, unroll=False)
      wait(0)
      scat(p, SB - 2, 0)
      wait(1)
      scat(p, SB - 1, 1)

    fire_idx(0, 0)

    def sb_pair(i, _):
      run_sb(2 * i, 0)
      run_sb(2 * i + 1, 1)
      return 0

    lax.fori_loop(0, chunks // SB // 2, sb_pair, 0, unroll=False)
    wait_idx(0)
    plsc.subcore_barrier()
    pltpu.sync_copy(acc.at[pl.ds(sid * rps, rps)],
                    out_hbm.at[cid, pl.ds(sid * rps, rps)])

  return agg_kernel


def _proj_scale_mm_body(xr_ref, wp_ref, bp_ref, do_ref, w1_ref, o_ref):
  x = jnp.dot(xr_ref[...], wp_ref[...], preferred_element_type=jnp.float32)
  x = jnp.maximum(x + bp_ref[...], 0.0)
  o_ref[...] = jnp.dot(x * _norm(do_ref), w1_ref[...],
                       preferred_element_type=jnp.float32)


def _norm(d_ref):
  deg = jnp.sum(d_ref[...], axis=0)
  return lax.rsqrt(jnp.maximum(deg, 1.0))[:, None]


def _mid_body(a0_ref, a1_ref, di_ref, do_ref, b_ref, w_ref, o_ref):
  h = jnp.maximum((a0_ref[...] + a1_ref[...]) * _norm(di_ref) + b_ref[...],
                  0.0)
  o_ref[...] = jnp.dot(h * _norm(do_ref), w_ref[...],
                       preferred_element_type=jnp.float32)


def _final_body(a0_ref, a1_ref, di_ref, b_ref, o_ref):
  o_ref[...] = jnp.maximum(
      (a0_ref[...] + a1_ref[...]) * _norm(di_ref) + b_ref[...], 0.0)


def _row_spec(d):
  return pl.BlockSpec((ROW_BLK, d), lambda i: (i, 0))


def _full_spec(shape):
  return pl.BlockSpec(shape, lambda i: tuple(0 for _ in shape))


def kernel(x_raw, edge_index, Wp, bp, W1, b1, W2, b2):
  n, in_dim = x_raw.shape
  hid = W1.shape[1]
  e = edge_index.shape[1]

  npad = ((n + ROW_BLK) // ROW_BLK) * ROW_BLK
  egrain = NW * CHUNK * SB * 2
  epad = ((e + egrain - 1) // egrain) * egrain
  n_dummy = npad - n

  src = edge_index[0].astype(jnp.int32)
  dst = edge_index[1].astype(jnp.int32)
  pad_idx = (jnp.arange(epad - e + SB * CHUNK, dtype=jnp.int32) % n_dummy) + n
  src = jnp.concatenate([src, pad_idx]).reshape(-1, CHUNK)
  dst = jnp.concatenate([dst, pad_idx]).reshape(-1, CHUNK)
  xp = jnp.pad(x_raw, ((0, npad - n), (0, 0)))

  grid = (npad // ROW_BLK,)

  deg_kernel = _make_degree_kernel(npad, epad)
  agg_kernel = _make_agg_kernel(npad, epad, hid)

  outd_p, ind_p = deg_kernel(src, dst)

  bp2 = bp.reshape(1, hid)
  b12 = b1.reshape(1, hid)
  b22 = b2.reshape(1, hid)
  dspec = pl.BlockSpec((NC, ROW_BLK), lambda i: (0, i))
  row = _row_spec(hid)
  wspec = _full_spec((in_dim, hid))
  bspec = _full_spec((1, hid))
  oshape = jax.ShapeDtypeStruct((npad, hid), jnp.float32)

  t1 = pl.pallas_call(
      _proj_scale_mm_body, grid=grid,
      in_specs=[pl.BlockSpec((ROW_BLK, in_dim), lambda i: (i, 0)), wspec,
                bspec, dspec, wspec],
      out_specs=row, out_shape=oshape,
  )(xp, Wp, bp2, outd_p, W1)

  agg1 = agg_kernel(t1, src, dst)

  t2 = pl.pallas_call(
      _mid_body, grid=grid,
      in_specs=[row, row, dspec, dspec, bspec, wspec],
      out_specs=row, out_shape=oshape,
  )(agg1[0], agg1[1], ind_p, outd_p, b12, W2)

  agg2 = agg_kernel(t2, src, dst)

  out = pl.pallas_call(
      _final_body, grid=grid,
      in_specs=[row, row, dspec, bspec],
      out_specs=row, out_shape=oshape,
  )(agg2[0], agg2[1], ind_p, b22)

  return out[:n]


# speedup vs baseline: 1.1562x; 1.0116x over previous
import functools

import jax
import jax.numpy as jnp
from jax import lax
from jax.experimental import pallas as pl
from jax.experimental.pallas import tpu as pltpu
from jax.experimental.pallas import tpu_sc as plsc

NC = 2
NS = 16
NW = NC * NS
CHUNK = 128
SB = 8
ROW_BLK = 1024


def _mesh():
  return plsc.VectorSubcoreMesh(core_axis_name="c", subcore_axis_name="s")


def _make_degree_kernel(npad, epad):
  epw = epad // NW
  chunks = epw // CHUNK
  rps = npad // NS

  @functools.partial(
      pl.kernel,
      out_type=(
          jax.ShapeDtypeStruct((NC, npad), jnp.float32),
          jax.ShapeDtypeStruct((NC, npad), jnp.float32),
      ),
      mesh=_mesh(),
      scratch_types=[
          pltpu.VMEM((chunks, CHUNK), jnp.int32),
          pltpu.VMEM((chunks, CHUNK), jnp.int32),
          pltpu.VMEM((CHUNK,), jnp.float32),
          pltpu.VMEM((rps,), jnp.float32),
          pltpu.VMEM_SHARED((npad,), jnp.float32),
          pltpu.VMEM_SHARED((npad,), jnp.float32),
          pltpu.SemaphoreType.DMA,
          pltpu.SemaphoreType.DMA,
      ],
  )
  def deg_kernel(src2_hbm, dst2_hbm, outd_hbm, ind_hbm, sidx, didx, ones,
                 zbuf, acc_o, acc_i, sem0, sem1):
    cid = lax.axis_index("c")
    sid = lax.axis_index("s")
    one = jnp.full((16,), 1.0, jnp.float32)
    zero = jnp.zeros((16,), jnp.float32)
    for k in range(CHUNK // 16):
      ones[pl.ds(k * 16, 16)] = one

    def zfill(i, _):
      zbuf[pl.ds(i * 16, 16)] = zero
      return 0

    lax.fori_loop(0, rps // 16, zfill, 0, unroll=False)
    pltpu.sync_copy(zbuf, acc_o.at[pl.ds(sid * rps, rps)])
    pltpu.sync_copy(zbuf, acc_i.at[pl.ds(sid * rps, rps)])

    rowbase = (cid * NS + sid) * chunks
    pltpu.sync_copy(src2_hbm.at[pl.ds(rowbase, chunks)], sidx)
    pltpu.sync_copy(dst2_hbm.at[pl.ds(rowbase, chunks)], didx)
    plsc.subcore_barrier()

    def body(g, _):
      pltpu.async_copy(ones, acc_o.at[sidx.at[g]], sem0, add=True)
      pltpu.async_copy(ones, acc_i.at[didx.at[g]], sem1, add=True)
      return 0

    lax.fori_loop(0, chunks, body, 0, unroll=False)

    def drain(g, _):
      pltpu.make_async_copy(ones, acc_o.at[sidx.at[0]], sem0).wait()
      pltpu.make_async_copy(ones, acc_i.at[didx.at[0]], sem1).wait()
      return 0

    lax.fori_loop(0, chunks, drain, 0, unroll=False)
    plsc.subcore_barrier()
    pltpu.sync_copy(acc_o.at[pl.ds(sid * rps, rps)],
                    outd_hbm.at[cid, pl.ds(sid * rps, rps)])
    pltpu.sync_copy(acc_i.at[pl.ds(sid * rps, rps)],
                    ind_hbm.at[cid, pl.ds(sid * rps, rps)])

  return deg_kernel


def _make_agg_kernel(npad, epad, d):
  epw = epad // NW
  chunks = epw // CHUNK
  rps = npad // NS

  @functools.partial(
      pl.kernel,
      out_type=jax.ShapeDtypeStruct((NC, npad, d), jnp.float32),
      mesh=_mesh(),
      scratch_types=[
          pltpu.VMEM((2, SB, CHUNK), jnp.int32),
          pltpu.VMEM((2, SB, CHUNK), jnp.int32),
          pltpu.VMEM((2, CHUNK, d), jnp.float32),
          pltpu.VMEM((64, d), jnp.float32),
          pltpu.VMEM_SHARED((npad, d), jnp.float32),
          pltpu.SemaphoreType.DMA,
          pltpu.SemaphoreType.DMA,
          pltpu.SemaphoreType.DMA,
      ],
  )
  def agg_kernel(t_hbm, src2_hbm, dst2_hbm, out_hbm, sidx, didx, rows, zbuf,
                 acc, sem0, sem1, isem):
    cid = lax.axis_index("c")
    sid = lax.axis_index("s")
    zero = jnp.zeros((16,), jnp.float32)

    def zfill(r, _):
      for j in range(d // 16):
        zbuf[r, pl.ds(j * 16, 16)] = zero
      return 0

    lax.fori_loop(0, 64, zfill, 0, unroll=False)

    def zero_fire(i, _):
      pltpu.async_copy(zbuf, acc.at[pl.ds(sid * rps + i * 64, 64)], isem)
      return 0

    def zero_drain(i, _):
      pltpu.make_async_copy(zbuf, acc.at[pl.ds(sid * rps, 64)], isem).wait()
      return 0

    lax.fori_loop(0, rps // 64, zero_fire, 0, unroll=False)
    lax.fori_loop(0, rps // 64, zero_drain, 0, unroll=False)
    plsc.subcore_barrier()

    rowbase = (cid * NS + sid) * chunks
    sems = (sem0, sem1)

    def fire_idx(sb, p):
      rowb = rowbase + sb * SB
      pltpu.async_copy(src2_hbm.at[pl.ds(rowb, SB)], sidx.at[p], isem)
      pltpu.async_copy(dst2_hbm.at[pl.ds(rowb, SB)], didx.at[p], isem)

    def wait_idx(p):
      pltpu.make_async_copy(src2_hbm.at[pl.ds(0, SB)], sidx.at[p],
                            isem).wait()
      pltpu.make_async_copy(dst2_hbm.at[pl.ds(0, SB)], didx.at[p],
                            isem).wait()

    def fire(p, c, b):
      pltpu.async_copy(t_hbm.at[sidx.at[p, c]], rows.at[b], sems[b])

    def wait(b):
      pltpu.make_async_copy(t_hbm.at[sidx.at[0, 0]], rows.at[b],
                            sems[b]).wait()

    def scat(p, c, b):
      pltpu.sync_copy(rows.at[b], acc.at[didx.at[p, c]], add=True)

    def run_sb(sb, p):
      wait_idx(p)
      fire_idx(sb + 1, 1 - p)
      fire(p, 0, 0)
      fire(p, 1, 1)

      def body(j, _):
        c0 = 2 * j
        wait(0)
        scat(p, c0, 0)
        fire(p, c0 + 2, 0)
        wait(1)
        scat(p, c0 + 1, 1)
        fire(p, c0 + 3, 1)
        return 0

      lax.fori_loop(0, (SB - 2) // 2, body, 0, unroll=False)
      wait(0)
      scat(p, SB - 2, 0)
      wait(1)
      scat(p, SB - 1, 1)

    fire_idx(0, 0)

    def sb_pair(i, _):
      run_sb(2 * i, 0)
      run_sb(2 * i + 1, 1)
      return 0

    lax.fori_loop(0, chunks // SB // 2, sb_pair, 0, unroll=False)
    wait_idx(0)
    plsc.subcore_barrier()
    pltpu.sync_copy(acc.at[pl.ds(sid * rps, rps)],
                    out_hbm.at[cid, pl.ds(sid * rps, rps)])

  return agg_kernel


def _proj_scale_mm_body(xr_ref, wp_ref, bp_ref, do_ref, w1_ref, o_ref):
  x = jnp.dot(xr_ref[...], wp_ref[...], preferred_element_type=jnp.float32)
  x = jnp.maximum(x + bp_ref[...], 0.0)
  o_ref[...] = jnp.dot(x * _norm(do_ref), w1_ref[...],
                       preferred_element_type=jnp.float32)


def _norm(d_ref):
  deg = jnp.sum(d_ref[...], axis=0)
  return lax.rsqrt(jnp.maximum(deg, 1.0))[:, None]


def _mid_body(a0_ref, a1_ref, di_ref, do_ref, b_ref, w_ref, o_ref):
  h = jnp.maximum((a0_ref[...] + a1_ref[...]) * _norm(di_ref) + b_ref[...],
                  0.0)
  o_ref[...] = jnp.dot(h * _norm(do_ref), w_ref[...],
                       preferred_element_type=jnp.float32)


def _final_body(a0_ref, a1_ref, di_ref, b_ref, o_ref):
  inorm = lax.rsqrt(jnp.maximum(jnp.sum(di_ref[...], axis=1), 1.0))[:, None]
  o_ref[...] = jnp.maximum(
      (a0_ref[...] + a1_ref[...]) * inorm + b_ref[...], 0.0)


def _row_spec(d):
  return pl.BlockSpec((ROW_BLK, d), lambda i: (i, 0))


def _full_spec(shape):
  return pl.BlockSpec(shape, lambda i: tuple(0 for _ in shape))


def kernel(x_raw, edge_index, Wp, bp, W1, b1, W2, b2):
  n, in_dim = x_raw.shape
  hid = W1.shape[1]
  e = edge_index.shape[1]

  npad = ((n + ROW_BLK) // ROW_BLK) * ROW_BLK
  egrain = NW * CHUNK * SB * 2
  epad = ((e + egrain - 1) // egrain) * egrain
  n_dummy = npad - n

  src = edge_index[0].astype(jnp.int32)
  dst = edge_index[1].astype(jnp.int32)
  pad_idx = (jnp.arange(epad - e + SB * CHUNK, dtype=jnp.int32) % n_dummy) + n
  src = jnp.concatenate([src, pad_idx]).reshape(-1, CHUNK)
  dst = jnp.concatenate([dst, pad_idx]).reshape(-1, CHUNK)
  xp = jnp.pad(x_raw, ((0, npad - n), (0, 0)))

  grid = (npad // ROW_BLK,)

  deg_kernel = _make_degree_kernel(npad, epad)
  agg_kernel = _make_agg_kernel(npad, epad, hid)

  outd_p, ind_p = deg_kernel(src, dst)

  bp2 = bp.reshape(1, hid)
  b12 = b1.reshape(1, hid)
  b22 = b2.reshape(1, hid)
  dspec = pl.BlockSpec((NC, ROW_BLK), lambda i: (0, i))
  row = _row_spec(hid)
  wspec = _full_spec((in_dim, hid))
  bspec = _full_spec((1, hid))
  oshape = jax.ShapeDtypeStruct((npad, hid), jnp.float32)

  t1 = pl.pallas_call(
      _proj_scale_mm_body, grid=grid,
      in_specs=[pl.BlockSpec((ROW_BLK, in_dim), lambda i: (i, 0)), wspec,
                bspec, dspec, wspec],
      out_specs=row, out_shape=oshape,
  )(xp, Wp, bp2, outd_p, W1)

  agg1 = agg_kernel(t1, src, dst)

  t2 = pl.pallas_call(
      _mid_body, grid=grid,
      in_specs=[row, row, dspec, dspec, bspec, wspec],
      out_specs=row, out_shape=oshape,
  )(agg1[0], agg1[1], ind_p, outd_p, b12, W2)

  agg2 = agg_kernel(t2, src, dst)

  fb = 1000 if n % 1000 == 0 else ROW_BLK
  fgrid = (n // fb,) if n % fb == 0 else grid
  frow = pl.BlockSpec((fb, hid), lambda i: (i, 0))
  fdspec = pl.BlockSpec((fb, NC), lambda i: (i, 0))
  fshape = jax.ShapeDtypeStruct((fgrid[0] * fb, hid), jnp.float32)
  out = pl.pallas_call(
      _final_body, grid=fgrid,
      in_specs=[frow, frow, fdspec, bspec],
      out_specs=frow, out_shape=fshape,
  )(agg2[0], agg2[1], ind_p.T, b22)

  return out[:n]
